# baseline (speedup 1.0000x reference)
import functools

import jax
import jax.numpy as jnp
from jax import lax
from jax.experimental import pallas as pl
from jax.experimental.pallas import tpu as pltpu
from jax.experimental.pallas import tpu_sc as plsc

NUM_EMB = 1_000_000
DIM = 32
BATCH = 16384

_info = plsc.get_sparse_core_info()
_NC, _NS = _info.num_cores, _info.num_subcores
_NW = _NC * _NS
_NCOL = NUM_EMB // 128
_TAIL0 = _NCOL * 128
_PW = 4
_MAXPASS = (_NCOL // _NW + 1 + _PW - 1) // _PW
_CAP = 640
_NPAD = 512
_BROWS = BATCH + _NPAD

_params = pltpu.CompilerParams(
    use_tc_tiling_on_sc=True, needs_layout_passes=False)
_mesh = plsc.VectorSubcoreMesh(core_axis_name="c", subcore_axis_name="s")


@functools.partial(
    pl.kernel,
    mesh=_mesh,
    compiler_params=_params,
    out_type=jax.ShapeDtypeStruct((_BROWS, 128), jnp.float32),
    scratch_types=[
        pltpu.VMEM((1024,), jnp.int32),
        pltpu.VMEM((_CAP + 16,), jnp.int32),
        pltpu.VMEM((_CAP + 16,), jnp.int32),
        pltpu.VMEM((_CAP + 16,), jnp.int32),
        pltpu.VMEM((2, DIM, _PW * 128), jnp.float32),
        pltpu.VMEM((_CAP + 16, 128), jnp.float32),
        pltpu.VMEM(((_CAP + 16) // 16, 16), jnp.int32),
        pltpu.VMEM((DIM, NUM_EMB - _TAIL0), jnp.float32),
        pltpu.SemaphoreType.DMA,
        pltpu.SemaphoreType.DMA,
        pltpu.SemaphoreType.DMA,
    ],
)
def _select(idx_hbm, table_hbm, tail_hbm, outc_hbm, idxb, sidx, sjj, jbuf,
            buf, rows, jc2, tailb, tsem0, tsem1, ssem):
    wid = lax.axis_index("s") * _NC + lax.axis_index("c")
    c0 = wid * _NCOL // _NW
    c1 = (wid + 1) * _NCOL // _NW
    ncols = c1 - c0
    npass = (ncols + _PW - 1) // _PW
    lanes = lax.iota(jnp.int32, 16)
    lo = c0 * 128
    hi = jnp.where(wid == _NW - 1, NUM_EMB, c1 * 128)
    tsems = (tsem0, tsem1)

    def tstart(p, b):
        rb = c0 + jnp.minimum(p * _PW, ncols - _PW)
        pltpu.async_copy(
            table_hbm.at[:, pl.ds(rb * 128, _PW * 128)], buf.at[b], tsems[b])

    tstart(0, 0)
    tstart(1, 1)

    def scan_stage(st, cnt):
        pltpu.sync_copy(idx_hbm.at[pl.ds(st * 1024, 1024)], idxb)

        def scan_chunk(k, cnt):
            chunk = idxb[pl.ds(k * 16, 16)]
            m = jnp.logical_and(chunk >= lo, chunk < hi)
            js = st * 1024 + k * 16 + lanes
            plsc.store_compressed(sidx.at[pl.ds(cnt, 16)], chunk, mask=m)
            plsc.store_compressed(sjj.at[pl.ds(cnt, 16)], js, mask=m)
            pc = lax.reduce_sum(jnp.where(m, 1, 0), axes=(0,))
            return jnp.minimum(cnt + pc, _CAP)

        return lax.fori_loop(0, 64, scan_chunk, cnt)

    cnt = lax.fori_loop(0, BATCH // 1024, scan_stage, 0)
    sidx[pl.ds(cnt, 16)] = jnp.full((16,), jnp.int32(0x7FFFFFF0))
    nchunks = (cnt + 15) // 16

    def extract(bufref, rbase, wlo, whi, cnt2):
        def chunk_body(k, cnt2):
            chunk = sidx[pl.ds(k * 16, 16)]
            m = jnp.logical_and(chunk >= wlo, chunk < whi)

            def has_work(state):
                m, _ = state
                return lax.reduce_or(m, axes=(0,))

            def one(state):
                m, cnt2 = state
                l = plsc.all_reduce_ffs(m)
                sel = lanes == l
                lc = lax.reduce_max(
                    jnp.where(sel, chunk - rbase, 0), axes=(0,))
                vlo = plsc.load_gather(bufref, [lanes, jnp.full((16,), lc)])
                vhi = plsc.load_gather(
                    bufref, [lanes + 16, jnp.full((16,), lc)])
                plsc.store_scatter(
                    rows, [jnp.full((16,), cnt2), lanes], vlo)
                plsc.store_scatter(
                    rows, [jnp.full((16,), cnt2), lanes + 16], vhi)
                return jnp.logical_and(m, jnp.logical_not(sel)), cnt2 + 1

            pc = lax.reduce_sum(jnp.where(m, 1, 0), axes=(0,))
            plsc.store_compressed(jbuf.at[pl.ds(cnt2, 16)],
                                  sjj[pl.ds(k * 16, 16)], mask=m)

            def go(cnt2):
                _, cnt2 = lax.while_loop(has_work, one, (m, cnt2))
                return cnt2

            return jnp.where(pc > 0, go(cnt2), cnt2)

        return lax.fori_loop(0, nchunks, chunk_body, cnt2)

    def flush(fired, upto):
        def fire(k, _):
            plsc.store_scatter(
                jc2, [jnp.full((16,), k), lanes], jbuf[pl.ds(k * 16, 16)])
            pltpu.async_copy(
                rows.at[pl.ds(k * 16, 16)],
                outc_hbm.at[jc2.at[k]],
                ssem,
            )
            return 0

        lax.fori_loop(fired, upto, fire, 0)
        return upto

    def pass_group(g, state):
        cnt2, fired = state
        for b in range(2):
            p = g * 2 + b

            def do(state, p=p, b=b):
                cnt2, fired = state
                rb = c0 + jnp.minimum(p * _PW, ncols - _PW)
                pltpu.make_async_copy(
                    table_hbm.at[:, pl.ds(rb * 128, _PW * 128)],
                    buf.at[b], tsems[b]).wait()
                wlo = (c0 + p * _PW) * 128
                whi = jnp.minimum(c0 + (p + 1) * _PW, c1) * 128
                cnt2 = extract(buf.at[b], rb * 128, wlo, whi, cnt2)

                @pl.when(p + 2 < npass)
                def _():
                    tstart(p + 2, b)

                fired = flush(fired, cnt2 // 16)
                return cnt2, fired

            cnt2, fired = lax.cond(p < npass, do, lambda s: s, (cnt2, fired))
        return cnt2, fired

    cnt2, fired = lax.fori_loop(0, (_MAXPASS + 1) // 2, pass_group, (0, 0))

    @pl.when(wid == _NW - 1)
    def _():
        pltpu.sync_copy(tail_hbm, tailb)

    cnt2 = jnp.where(
        wid == _NW - 1,
        extract(tailb, _TAIL0, _TAIL0, NUM_EMB, cnt2),
        cnt2,
    )

    jbuf[pl.ds(cnt2, 16)] = BATCH + (wid * 16 + lanes)
    nfinal = (cnt2 + 15) // 16
    fired = flush(fired, nfinal)

    def drain(k, _):
        pltpu.make_async_copy(
            rows.at[pl.ds(0, 16)],
            outc_hbm.at[jc2.at[0]],
            ssem,
        ).wait()
        return 0

    lax.fori_loop(0, nfinal, drain, 0)


@functools.partial(
    pl.kernel,
    mesh=_mesh,
    compiler_params=_params,
    out_type=jax.ShapeDtypeStruct((DIM, BATCH), jnp.float32),
    scratch_types=[
        pltpu.VMEM((128, 128), jnp.float32),
        pltpu.VMEM((DIM, 128), jnp.float32),
    ],
)
def _transpose(outc_hbm, out_hbm, tbuf, obuf):
    wid = lax.axis_index("s") * _NC + lax.axis_index("c")
    lanes = lax.iota(jnp.int32, 16)
    for q in range(BATCH // 128 // _NW):
        blk = wid * (BATCH // 128 // _NW) + q
        pltpu.sync_copy(outc_hbm.at[pl.ds(blk * 128, 128), :], tbuf)
        for c in range(DIM):
            for j16 in range(8):
                v = plsc.load_gather(
                    tbuf, [j16 * 16 + lanes, jnp.full((16,), c)])
                obuf[c, pl.ds(j16 * 16, 16)] = v
        pltpu.sync_copy(obuf, out_hbm.at[:, pl.ds(blk * 128, 128)])


@jax.jit
def kernel(indices, table):
    tail = lax.slice(table, (_TAIL0, 0), (NUM_EMB, DIM)).T
    outc = _select(indices.astype(jnp.int32), table.T, tail)
    out_t = _transpose(outc)
    return out_t.T


# speedup vs baseline: 1.0241x; 1.0241x over previous
import functools

import jax
import jax.numpy as jnp
from jax import lax
from jax.experimental import pallas as pl
from jax.experimental.pallas import tpu as pltpu
from jax.experimental.pallas import tpu_sc as plsc

NUM_EMB = 1_000_000
DIM = 32
BATCH = 16384

_info = plsc.get_sparse_core_info()
_NC, _NS = _info.num_cores, _info.num_subcores
_NW = _NC * _NS
_NCOL = NUM_EMB // 128
_TAIL0 = _NCOL * 128
_PW = 4
_MAXPASS = (_NCOL // _NW + 1 + _PW - 1) // _PW
_CAP = 640
_NPAD = 512
_BROWS = BATCH + _NPAD

_params = pltpu.CompilerParams(
    use_tc_tiling_on_sc=True, needs_layout_passes=False)
_mesh = plsc.VectorSubcoreMesh(core_axis_name="c", subcore_axis_name="s")


@functools.partial(
    pl.kernel,
    mesh=_mesh,
    compiler_params=_params,
    out_type=jax.ShapeDtypeStruct((_BROWS, 128), jnp.float32),
    scratch_types=[
        pltpu.VMEM((1024,), jnp.int32),
        pltpu.VMEM((_CAP + 16,), jnp.int32),
        pltpu.VMEM((_CAP + 16,), jnp.int32),
        pltpu.VMEM((_CAP + 16,), jnp.int32),
        pltpu.VMEM((2, DIM, _PW * 128), jnp.float32),
        pltpu.VMEM((_CAP + 16, 128), jnp.float32),
        pltpu.VMEM(((_CAP + 16) // 16, 16), jnp.int32),
        pltpu.VMEM((DIM, NUM_EMB - _TAIL0), jnp.float32),
        pltpu.SemaphoreType.DMA,
        pltpu.SemaphoreType.DMA,
        pltpu.SemaphoreType.DMA,
    ],
)
def _select(idx_hbm, table_hbm, tail_hbm, outc_hbm, idxb, sidx, sjj, jbuf,
            buf, rows, jc2, tailb, tsem0, tsem1, ssem):
    wid = lax.axis_index("s") * _NC + lax.axis_index("c")
    c0 = wid * _NCOL // _NW
    c1 = (wid + 1) * _NCOL // _NW
    ncols = c1 - c0
    npass = (ncols + _PW - 1) // _PW
    lanes = lax.iota(jnp.int32, 16)
    lo = c0 * 128
    hi = jnp.where(wid == _NW - 1, NUM_EMB, c1 * 128)
    tsems = (tsem0, tsem1)

    def tstart(p, b):
        rb = c0 + jnp.minimum(p * _PW, ncols - _PW)
        pltpu.async_copy(
            table_hbm.at[:, pl.ds(rb * 128, _PW * 128)], buf.at[b], tsems[b])

    tstart(0, 0)
    tstart(1, 1)

    def scan_stage(st, cnt):
        pltpu.sync_copy(idx_hbm.at[pl.ds(st * 1024, 1024)], idxb)

        def scan_chunk(k, cnt):
            chunk = idxb[pl.ds(k * 16, 16)]
            m = jnp.logical_and(chunk >= lo, chunk < hi)
            js = st * 1024 + k * 16 + lanes
            plsc.store_compressed(sidx.at[pl.ds(cnt, 16)], chunk, mask=m)
            plsc.store_compressed(sjj.at[pl.ds(cnt, 16)], js, mask=m)
            pc = lax.reduce_sum(jnp.where(m, 1, 0), axes=(0,))
            return jnp.minimum(cnt + pc, _CAP)

        return lax.fori_loop(0, 64, scan_chunk, cnt)

    cnt = lax.fori_loop(0, BATCH // 1024, scan_stage, 0)
    sidx[pl.ds(cnt, 16)] = jnp.full((16,), jnp.int32(0x7FFFFFF0))
    nchunks = (cnt + 15) // 16

    def extract(bufref, rbase, wlo, whi, cnt2):
        def chunk_body(k, cnt2):
            chunk = sidx[pl.ds(k * 16, 16)]
            m = jnp.logical_and(chunk >= wlo, chunk < whi)

            def has_work(state):
                m, _ = state
                return lax.reduce_or(m, axes=(0,))

            def one(state):
                m, cnt2 = state
                l = plsc.all_reduce_ffs(m)
                sel = lanes == l
                lc = lax.reduce_max(
                    jnp.where(sel, chunk - rbase, 0), axes=(0,))
                vlo = plsc.load_gather(bufref, [lanes, jnp.full((16,), lc)])
                vhi = plsc.load_gather(
                    bufref, [lanes + 16, jnp.full((16,), lc)])
                plsc.store_scatter(
                    rows, [jnp.full((16,), cnt2), lanes], vlo)
                plsc.store_scatter(
                    rows, [jnp.full((16,), cnt2), lanes + 16], vhi)
                return jnp.logical_and(m, jnp.logical_not(sel)), cnt2 + 1

            plsc.store_compressed(jbuf.at[pl.ds(cnt2, 16)],
                                  sjj[pl.ds(k * 16, 16)], mask=m)
            _, cnt2 = lax.while_loop(has_work, one, (m, cnt2))
            return cnt2

        return lax.fori_loop(0, nchunks, chunk_body, cnt2)

    def flush(fired, upto):
        def fire(k, _):
            plsc.store_scatter(
                jc2, [jnp.full((16,), k), lanes], jbuf[pl.ds(k * 16, 16)])
            pltpu.async_copy(
                rows.at[pl.ds(k * 16, 16)],
                outc_hbm.at[jc2.at[k]],
                ssem,
            )
            return 0

        lax.fori_loop(fired, upto, fire, 0)
        return upto

    def pass_group(g, state):
        cnt2, fired = state
        for b in range(2):
            p = g * 2 + b

            def do(state, p=p, b=b):
                cnt2, fired = state
                rb = c0 + jnp.minimum(p * _PW, ncols - _PW)
                pltpu.make_async_copy(
                    table_hbm.at[:, pl.ds(rb * 128, _PW * 128)],
                    buf.at[b], tsems[b]).wait()
                wlo = (c0 + p * _PW) * 128
                whi = jnp.minimum(c0 + (p + 1) * _PW, c1) * 128
                cnt2 = extract(buf.at[b], rb * 128, wlo, whi, cnt2)

                @pl.when(p + 2 < npass)
                def _():
                    tstart(p + 2, b)

                fired = flush(fired, cnt2 // 16)
                return cnt2, fired

            cnt2, fired = lax.cond(p < npass, do, lambda s: s, (cnt2, fired))
        return cnt2, fired

    cnt2, fired = lax.fori_loop(0, (_MAXPASS + 1) // 2, pass_group, (0, 0))

    @pl.when(wid == _NW - 1)
    def _():
        pltpu.sync_copy(tail_hbm, tailb)

    cnt2 = jnp.where(
        wid == _NW - 1,
        extract(tailb, _TAIL0, _TAIL0, NUM_EMB, cnt2),
        cnt2,
    )

    jbuf[pl.ds(cnt2, 16)] = BATCH + (wid * 16 + lanes)
    nfinal = (cnt2 + 15) // 16
    fired = flush(fired, nfinal)

    def drain(k, _):
        pltpu.make_async_copy(
            rows.at[pl.ds(0, 16)],
            outc_hbm.at[jc2.at[0]],
            ssem,
        ).wait()
        return 0

    lax.fori_loop(0, nfinal, drain, 0)


@functools.partial(
    pl.kernel,
    mesh=_mesh,
    compiler_params=_params,
    out_type=jax.ShapeDtypeStruct((DIM, BATCH), jnp.float32),
    scratch_types=[
        pltpu.VMEM((2, 128, 128), jnp.float32),
        pltpu.VMEM((2, DIM, 128), jnp.float32),
        pltpu.SemaphoreType.DMA,
        pltpu.SemaphoreType.DMA,
        pltpu.SemaphoreType.DMA,
        pltpu.SemaphoreType.DMA,
    ],
)
def _transpose(outc_hbm, out_hbm, tbuf, obuf, isem0, isem1, osem0, osem1):
    wid = lax.axis_index("s") * _NC + lax.axis_index("c")
    lanes = lax.iota(jnp.int32, 16)
    isems = (isem0, isem1)
    osems = (osem0, osem1)
    nblk = BATCH // 128 // _NW

    def iref(q):
        blk = wid * nblk + q
        return outc_hbm.at[pl.ds(blk * 128, 128), :]

    def oref(q):
        blk = wid * nblk + q
        return out_hbm.at[:, pl.ds(blk * 128, 128)]

    pltpu.async_copy(iref(0), tbuf.at[0], isems[0])
    pltpu.async_copy(iref(1), tbuf.at[1], isems[1])
    for q in range(nblk):
        b = q % 2
        pltpu.make_async_copy(iref(q), tbuf.at[b], isems[b]).wait()
        if q >= 2:
            pltpu.make_async_copy(obuf.at[b], oref(q - 2), osems[b]).wait()
        for c in range(DIM):
            for j16 in range(8):
                v = plsc.load_gather(
                    tbuf.at[b], [j16 * 16 + lanes, jnp.full((16,), c)])
                obuf[b, c, pl.ds(j16 * 16, 16)] = v
        pltpu.async_copy(obuf.at[b], oref(q), osems[b])
        if q + 2 < nblk:
            pltpu.async_copy(iref(q + 2), tbuf.at[b], isems[b])
    for q in range(nblk - 2, nblk):
        b = q % 2
        pltpu.make_async_copy(obuf.at[b], oref(q), osems[b]).wait()


@jax.jit
def kernel(indices, table):
    tail = lax.slice(table, (_TAIL0, 0), (NUM_EMB, DIM)).T
    outc = _select(indices.astype(jnp.int32), table.T, tail)
    out_t = _transpose(outc)
    return out_t.T
